# Initial kernel scaffold; baseline (speedup 1.0000x reference)
import jax
import jax.numpy as jnp
from jax.experimental import pallas as pl


def kernel(input_ids, word_embeddings_weight, position_embeddings_weight):
    raise NotImplementedError("write your pallas kernel here")


# speedup vs baseline: 1.0289x; 1.0289x over previous
import functools

import jax
import jax.numpy as jnp
from jax import lax
from jax.experimental import pallas as pl
from jax.experimental.pallas import tpu as pltpu
from jax.experimental.pallas import tpu_sc as plsc

B = 4
S = 8192
D = 768
LANES = 16

NC = 2
NS = 16
NW = NC * NS

TOK = B * S
TPW = TOK // NW
CHUNK = 64
NCHUNK = TPW // CHUNK
GROUPS = D // LANES


def _body(ids_hbm, word_hbm, pos_hbm, out_hbm, idx_v, rows_v, pos_v, sem):
    wid = lax.axis_index("s") * NC + lax.axis_index("c")
    base = wid * TPW
    poff = base % S

    pltpu.sync_copy(ids_hbm.at[wid], idx_v)

    def chunk_body(c, carry):
        gather = pltpu.async_copy(word_hbm.at[idx_v.at[c]], rows_v, sem)
        pltpu.sync_copy(pos_hbm.at[pl.ds(poff + c * CHUNK, CHUNK)], pos_v)
        gather.wait()

        def row_body(r, rcarry):
            for j in range(GROUPS):
                sl = pl.ds(j * LANES, LANES)
                rows_v[r, sl] = rows_v[r, sl] + pos_v[r, sl]
            return rcarry

        lax.fori_loop(0, CHUNK, row_body, 0, unroll=False)

        pltpu.sync_copy(rows_v, out_hbm.at[pl.ds(base + c * CHUNK, CHUNK)])
        return carry

    lax.fori_loop(0, NCHUNK, chunk_body, 0, unroll=False)


@jax.jit
def kernel(input_ids, word_embeddings_weight, position_embeddings_weight):
    ids = jnp.reshape(input_ids.astype(jnp.int32), (NW, NCHUNK, CHUNK))
    fn = pl.kernel(
        _body,
        out_type=jax.ShapeDtypeStruct((TOK, D), jnp.float32),
        mesh=plsc.VectorSubcoreMesh(core_axis_name="c", subcore_axis_name="s"),
        scratch_types=[
            pltpu.VMEM((NCHUNK, CHUNK), jnp.int32),
            pltpu.VMEM((CHUNK, D), jnp.float32),
            pltpu.VMEM((CHUNK, D), jnp.float32),
            pltpu.SemaphoreType.DMA,
        ],
    )
    out = fn(ids, word_embeddings_weight, position_embeddings_weight)
    return jnp.reshape(out, (B, S, D))


# speedup vs baseline: 1.6347x; 1.5889x over previous
import jax
import jax.numpy as jnp
from jax import lax
from jax.experimental import pallas as pl
from jax.experimental.pallas import tpu as pltpu
from jax.experimental.pallas import tpu_sc as plsc

B = 4
S = 8192
D = 768
LANES = 16

NC = 2
NS = 16
NW = NC * NS

SPW = S // NW
C = 32
NSC = SPW // C
NUNIT = NSC * B
UPT = 8
NT = NUNIT // UPT
GROUPS = D // LANES


def _body(ids_hbm, word_hbm, pos_hbm, out_hbm,
          idx_v, rows0, rows1, pos0, pos1,
          gsem0, gsem1, psem0, psem1, wsem0, wsem1):
    wid = lax.axis_index("s") * NC + lax.axis_index("c")
    soff = wid * SPW

    rows = (rows0, rows1)
    pos = (pos0, pos1)
    gsem = (gsem0, gsem1)
    psem = (psem0, psem1)
    wsem = (wsem0, wsem1)

    pltpu.sync_copy(ids_hbm.at[wid], idx_v)

    def issue_pos(sc, q):
        pltpu.async_copy(pos_hbm.at[pl.ds(soff + sc * C, C)], pos[q], psem[q])

    def wait_pos(q):
        pltpu.make_async_copy(pos_hbm.at[pl.ds(0, C)], pos[q], psem[q]).wait()

    def issue_gather(sc, b, p):
        pltpu.async_copy(word_hbm.at[idx_v.at[sc, b]], rows[p], gsem[p])

    def wait_gather(p):
        pltpu.make_async_copy(
            word_hbm.at[idx_v.at[0, 0]], rows[p], gsem[p]).wait()

    def issue_write(sc, b, p):
        pltpu.async_copy(
            rows[p], out_hbm.at[pl.ds(b * S + soff + sc * C, C)], wsem[p])

    def wait_write(p):
        pltpu.make_async_copy(
            rows[p], out_hbm.at[pl.ds(0, C)], wsem[p]).wait()

    def add_pos(p, q):
        rbuf = rows[p]
        pbuf = pos[q]

        def row_body(r, carry):
            for j in range(GROUPS):
                sl = pl.ds(j * LANES, LANES)
                plsc.addupdate(rbuf.at[r, sl], pbuf[r, sl])
            return carry

        lax.fori_loop(0, C, row_body, 0, unroll=False)

    issue_pos(0, 0)
    issue_gather(0, 0, 0)

    def iter_body(t, carry):
        for k in range(UPT):
            p = k % 2
            q = k // 4
            sc = 2 * t + q
            b = k % 4
            if k == 0:
                issue_pos(sc + 1, 1)
                wait_pos(0)
            if k == 4:
                @pl.when(t < NT - 1)
                def _():
                    issue_pos(sc + 1, 0)
                wait_pos(1)
            if k == 0:
                @pl.when(t > 0)
                def _():
                    wait_write(1 - p)
                issue_gather(sc, b + 1, 1 - p)
            elif k == UPT - 1:
                wait_write(1 - p)
                @pl.when(t < NT - 1)
                def _():
                    issue_gather(sc + 1, 0, 1 - p)
            else:
                wait_write(1 - p)
                issue_gather(sc + (1 if k == 3 else 0), (b + 1) % 4, 1 - p)
            wait_gather(p)
            add_pos(p, q)
            issue_write(sc, b, p)
        return carry

    lax.fori_loop(0, NT, iter_body, 0, unroll=False)

    wait_write(1)


@jax.jit
def kernel(input_ids, word_embeddings_weight, position_embeddings_weight):
    ids = jnp.transpose(
        jnp.reshape(input_ids.astype(jnp.int32), (B, NW, NSC, C)),
        (1, 2, 0, 3))
    fn = pl.kernel(
        _body,
        out_type=jax.ShapeDtypeStruct((B * S, D), jnp.float32),
        mesh=plsc.VectorSubcoreMesh(core_axis_name="c", subcore_axis_name="s"),
        scratch_types=[
            pltpu.VMEM((NSC, B, C), jnp.int32),
            pltpu.VMEM((C, D), jnp.float32),
            pltpu.VMEM((C, D), jnp.float32),
            pltpu.VMEM((C, D), jnp.float32),
            pltpu.VMEM((C, D), jnp.float32),
            pltpu.SemaphoreType.DMA,
            pltpu.SemaphoreType.DMA,
            pltpu.SemaphoreType.DMA,
            pltpu.SemaphoreType.DMA,
            pltpu.SemaphoreType.DMA,
            pltpu.SemaphoreType.DMA,
        ],
    )
    out = fn(ids, word_embeddings_weight, position_embeddings_weight)
    return jnp.reshape(out, (B, S, D))
